# baseline (speedup 1.0000x reference)
import functools

import jax
import jax.numpy as jnp
from jax import lax
from jax.experimental import pallas as pl
from jax.experimental.pallas import tpu as pltpu
from jax.experimental.pallas import tpu_sc as plsc

N = 100000
D = 128

_INFO = plsc.get_sparse_core_info()
_NC = _INFO.num_cores
_NS = _INFO.num_subcores
_NW = _NC * _NS

V = 512

PER_W = 3128
CS = 128
NCH = 25
STAGE = NCH * CS
NBUF = 6


def _make_kernel():
    mesh = plsc.VectorSubcoreMesh(core_axis_name="c", subcore_axis_name="s")
    scratch = [pltpu.VMEM((STAGE,), jnp.int32),
               pltpu.VMEM_SHARED((V, D), jnp.float32),
               pltpu.VMEM((NBUF * CS, D), jnp.float32),
               pltpu.SemaphoreType.DMA((NBUF,)),
               pltpu.SemaphoreType.DMA((NBUF,)),
               pltpu.SemaphoreType.DMA]

    @functools.partial(
        pl.kernel,
        mesh=mesh,
        out_type=jax.ShapeDtypeStruct((N, D), jnp.float32),
        scratch_types=scratch,
    )
    def emb(table_hbm, idx_hbm, out_hbm, idx_v, table_sh, rows_v, gsems, ssems,
            tsem):
        rows = [rows_v.at[pl.ds(b * CS, CS)] for b in range(NBUF)]
        gsem = [gsems.at[b] for b in range(NBUF)]
        ssem = [ssems.at[b] for b in range(NBUF)]

        sid = lax.axis_index("s")
        wid = sid * _NC + lax.axis_index("c")
        base = wid * PER_W

        seg = V // _NS
        tcp = pltpu.async_copy(table_hbm.at[pl.ds(sid * seg, seg)],
                               table_sh.at[pl.ds(sid * seg, seg)], tsem)
        stage0 = jnp.minimum(base, N - STAGE)
        pltpu.sync_copy(idx_hbm.at[pl.ds(stage0, STAGE)], idx_v)
        tcp.wait()
        plsc.subcore_barrier()

        r0 = [jnp.minimum(base + c * CS, N - CS) for c in range(NCH)]

        def gather(c, b):
            return pltpu.async_copy(
                table_sh.at[idx_v.at[pl.ds(r0[c] - stage0, CS)]],
                rows[b], gsem[b])

        def store(c, b):
            return pltpu.async_copy(
                rows[b], out_hbm.at[pl.ds(r0[c], CS)], ssem[b])

        warm = NBUF - 1
        gh, sh = {}, {}
        waited = set()
        for c in range(min(warm, NCH)):
            gh[c] = gather(c, c % NBUF)
        for c in range(NCH):
            b = c % NBUF
            j = c + warm
            if j < NCH:
                if j >= NBUF:
                    sh[j - NBUF].wait()
                    waited.add(j - NBUF)
                gh[j] = gather(j, j % NBUF)
            gh[c].wait()
            sh[c] = store(c, b)
        for c in range(NCH):
            if c not in waited:
                sh[c].wait()

    return emb


_EMB = _make_kernel()


def kernel(node_types, node_type_embeddings):
    return _EMB(node_type_embeddings, node_types)


# speedup vs baseline: 1.0044x; 1.0044x over previous
import functools

import jax
import jax.numpy as jnp
from jax import lax
from jax.experimental import pallas as pl
from jax.experimental.pallas import tpu as pltpu
from jax.experimental.pallas import tpu_sc as plsc

N = 100000
D = 128

_INFO = plsc.get_sparse_core_info()
_NC = _INFO.num_cores
_NS = _INFO.num_subcores
_NW = _NC * _NS

V = 512

PER_W = 3128
CS = 128
NCH = 25
STAGE = NCH * CS
NBUF = 4


def _make_kernel():
    mesh = plsc.VectorSubcoreMesh(core_axis_name="c", subcore_axis_name="s")
    scratch = [pltpu.VMEM((STAGE,), jnp.int32),
               pltpu.VMEM_SHARED((V, D), jnp.float32),
               pltpu.VMEM((NBUF * CS, D), jnp.float32),
               pltpu.SemaphoreType.DMA((NBUF,)),
               pltpu.SemaphoreType.DMA((NBUF,)),
               pltpu.SemaphoreType.DMA]

    @functools.partial(
        pl.kernel,
        mesh=mesh,
        out_type=jax.ShapeDtypeStruct((N, D), jnp.float32),
        scratch_types=scratch,
    )
    def emb(table_hbm, idx_hbm, out_hbm, idx_v, table_sh, rows_v, gsems, ssems,
            tsem):
        rows = [rows_v.at[pl.ds(b * CS, CS)] for b in range(NBUF)]
        gsem = [gsems.at[b] for b in range(NBUF)]
        ssem = [ssems.at[b] for b in range(NBUF)]

        sid = lax.axis_index("s")
        wid = sid * _NC + lax.axis_index("c")
        base = wid * PER_W

        seg = V // _NS
        tcp = pltpu.async_copy(table_hbm.at[pl.ds(sid * seg, seg)],
                               table_sh.at[pl.ds(sid * seg, seg)], tsem)
        stage0 = jnp.minimum(base, N - STAGE)
        pltpu.sync_copy(idx_hbm.at[pl.ds(stage0, STAGE)], idx_v)
        tcp.wait()
        plsc.subcore_barrier()

        r0 = [jnp.minimum(base + c * CS, N - CS) for c in range(NCH)]

        def gather(c, b):
            return pltpu.async_copy(
                table_sh.at[idx_v.at[pl.ds(r0[c] - stage0, CS)]],
                rows[b], gsem[b])

        def store(c, b):
            return pltpu.async_copy(
                rows[b], out_hbm.at[pl.ds(r0[c], CS)], ssem[b])

        warm = NBUF - 1
        gh, sh = {}, {}
        waited = set()
        for c in range(min(warm, NCH)):
            gh[c] = gather(c, c % NBUF)
        for c in range(NCH):
            b = c % NBUF
            j = c + warm
            if j < NCH:
                if j >= NBUF:
                    sh[j - NBUF].wait()
                    waited.add(j - NBUF)
                gh[j] = gather(j, j % NBUF)
            gh[c].wait()
            sh[c] = store(c, b)
        for c in range(NCH):
            if c not in waited:
                sh[c].wait()

    return emb


_EMB = _make_kernel()


def kernel(node_types, node_type_embeddings):
    return _EMB(node_type_embeddings, node_types)


# speedup vs baseline: 1.0068x; 1.0024x over previous
import functools

import jax
import jax.numpy as jnp
from jax import lax
from jax.experimental import pallas as pl
from jax.experimental.pallas import tpu as pltpu
from jax.experimental.pallas import tpu_sc as plsc

N = 100000
D = 128

_INFO = plsc.get_sparse_core_info()
_NC = _INFO.num_cores
_NS = _INFO.num_subcores
_NW = _NC * _NS

V = 512

CS = 128
NBUF = 4
PER_W = (-(-N // _NW) + 7) // 8 * 8
NCH = -(-PER_W // CS)
STAGE = NCH * CS


def _make_kernel():
    mesh = plsc.VectorSubcoreMesh(core_axis_name="c", subcore_axis_name="s")
    scratch = [pltpu.VMEM((STAGE,), jnp.int32),
               pltpu.VMEM_SHARED((V, D), jnp.float32),
               pltpu.VMEM((NBUF * CS, D), jnp.float32),
               pltpu.SemaphoreType.DMA((NBUF,)),
               pltpu.SemaphoreType.DMA((NBUF,)),
               pltpu.SemaphoreType.DMA]

    @functools.partial(
        pl.kernel,
        mesh=mesh,
        out_type=jax.ShapeDtypeStruct((N, D), jnp.float32),
        scratch_types=scratch,
    )
    def emb(table_hbm, idx_hbm, out_hbm, idx_v, table_sh, rows_v, gsems, ssems,
            tsem):
        rows = [rows_v.at[pl.ds(b * CS, CS)] for b in range(NBUF)]
        gsem = [gsems.at[b] for b in range(NBUF)]
        ssem = [ssems.at[b] for b in range(NBUF)]

        sid = lax.axis_index("s")
        wid = sid * _NC + lax.axis_index("c")
        base = wid * PER_W

        seg = V // _NS
        tcp = pltpu.async_copy(table_hbm.at[pl.ds(sid * seg, seg)],
                               table_sh.at[pl.ds(sid * seg, seg)], tsem)
        stage0 = jnp.minimum(base, N - STAGE)
        pltpu.sync_copy(idx_hbm.at[pl.ds(stage0, STAGE)], idx_v)
        tcp.wait()
        plsc.subcore_barrier()

        r0 = [jnp.minimum(base + c * CS, N - CS) for c in range(NCH)]

        def gather(c, b):
            return pltpu.async_copy(
                table_sh.at[idx_v.at[pl.ds(r0[c] - stage0, CS)]],
                rows[b], gsem[b])

        def store(c, b):
            return pltpu.async_copy(
                rows[b], out_hbm.at[pl.ds(r0[c], CS)], ssem[b])

        warm = NBUF - 1
        gh, sh = {}, {}
        waited = set()
        for c in range(min(warm, NCH)):
            gh[c] = gather(c, c % NBUF)
        for c in range(NCH):
            b = c % NBUF
            j = c + warm
            if j < NCH:
                if j >= NBUF:
                    sh[j - NBUF].wait()
                    waited.add(j - NBUF)
                gh[j] = gather(j, j % NBUF)
            gh[c].wait()
            sh[c] = store(c, b)
        for c in range(NCH):
            if c not in waited:
                sh[c].wait()

    return emb


_EMB = _make_kernel()


def kernel(node_types, node_type_embeddings):
    return _EMB(node_type_embeddings, node_types)
